# Initial kernel scaffold; baseline (speedup 1.0000x reference)
import jax
import jax.numpy as jnp
from jax.experimental import pallas as pl


def kernel(features, labels, centers):
    raise NotImplementedError("write your pallas kernel here")


# speedup vs baseline: 3.7940x; 3.7940x over previous
import functools

import jax
import jax.numpy as jnp
from jax import lax
from jax.experimental import pallas as pl
from jax.experimental.pallas import tpu as pltpu
from jax.experimental.pallas import tpu_sc as plsc

MARGIN = 2.5
LAMBDA_SPREAD = 0.5

NC = 2
NS = 16
L = 16


def _fill2d(ref, value):
  rows, cols = ref.shape
  vec = jnp.full((L,), value, dtype=jnp.float32)

  def body(r, carry):
    for c in range(cols // L):
      ref[r, pl.ds(c * L, L)] = vec
    return carry

  lax.fori_loop(0, rows, body, 0)


def _sc_segment_sums(features, labels, k_pad):
  batch, dim = features.shape
  nw = NC * NS
  b_per_w = batch // nw
  sub = 128
  nsub = b_per_w // sub
  stripe = k_pad // NS

  mesh = plsc.VectorSubcoreMesh(core_axis_name="c", subcore_axis_name="s")
  f32 = jnp.float32

  @functools.partial(
      pl.kernel,
      out_type=[
          jax.ShapeDtypeStruct((NC, k_pad, dim), f32),
          jax.ShapeDtypeStruct((NC, k_pad, dim), f32),
          jax.ShapeDtypeStruct((NC, k_pad, dim), f32),
      ],
      mesh=mesh,
      scratch_types=[
          pltpu.VMEM((stripe, dim), f32),
          pltpu.VMEM((sub, dim), f32),
          pltpu.VMEM((sub, dim), f32),
          pltpu.VMEM((sub, dim), f32),
      ] + [pltpu.VMEM((sub,), jnp.int32) for _ in range(nsub)] + [
          pltpu.VMEM_SHARED((k_pad, dim), f32),
          pltpu.VMEM_SHARED((k_pad, dim), f32),
          pltpu.VMEM_SHARED((k_pad, dim), f32),
      ],
  )
  def sc(feats_hbm, labels_hbm, s_out, s2_out, a_out,
         zbuf, fbuf, sqbuf, ones_v, *rest):
    lab_refs = rest[:nsub]
    shared_s, shared_s2, shared_a = rest[nsub:]
    cid = lax.axis_index("c")
    sid = lax.axis_index("s")

    _fill2d(zbuf, 0.0)
    _fill2d(ones_v, 1.0)

    row0 = sid * stripe
    pltpu.sync_copy(zbuf, shared_s.at[pl.ds(row0, stripe)])
    pltpu.sync_copy(zbuf, shared_s2.at[pl.ds(row0, stripe)])
    pltpu.sync_copy(zbuf, shared_a.at[pl.ds(row0, stripe)])
    plsc.subcore_barrier()

    base = (cid * NS + sid) * b_per_w
    for j in range(nsub):
      off = base + j * sub
      pltpu.sync_copy(labels_hbm.at[pl.ds(off, sub)], lab_refs[j])
      pltpu.sync_copy(feats_hbm.at[pl.ds(off, sub)], fbuf)

      def sq_body(r, carry):
        for c in range(dim // L):
          v = fbuf[r, pl.ds(c * L, L)]
          sqbuf[r, pl.ds(c * L, L)] = v * v
        return carry

      lax.fori_loop(0, sub, sq_body, 0)

      pltpu.sync_copy(fbuf, shared_s.at[lab_refs[j]], add=True)
      pltpu.sync_copy(sqbuf, shared_s2.at[lab_refs[j]], add=True)
      pltpu.sync_copy(ones_v, shared_a.at[lab_refs[j]], add=True)

    plsc.subcore_barrier()
    pltpu.sync_copy(shared_s.at[pl.ds(row0, stripe)],
                    s_out.at[cid, pl.ds(row0, stripe)])
    pltpu.sync_copy(shared_s2.at[pl.ds(row0, stripe)],
                    s2_out.at[cid, pl.ds(row0, stripe)])
    pltpu.sync_copy(shared_a.at[pl.ds(row0, stripe)],
                    a_out.at[cid, pl.ds(row0, stripe)])

  return sc(features, labels)


def _tc_combine(centers, s, s2, a, k_pad):
  k, dim = centers.shape
  f32 = jnp.float32

  def body(c_ref, s_ref, s2_ref, a_ref, out_ref):
    cmat = c_ref[:]
    smat = (s_ref[0] + s_ref[1])[:k]
    s2mat = (s2_ref[0] + s2_ref[1])[:k]
    nvec = (a_ref[0] + a_ref[1])[:k, 0:1]

    q = jnp.sum(s2mat, axis=1, keepdims=True)
    cdots = jnp.sum(cmat * smat, axis=1, keepdims=True)
    sq = jnp.sum(cmat * cmat, axis=1, keepdims=True)

    present = nvec > 0.0
    pcount = jnp.sum(present.astype(f32))
    safe_n = jnp.where(present, nvec, 1.0)
    lc_terms = jnp.where(present, (q - 2.0 * cdots + nvec * sq) / safe_n, 0.0)
    l_center = jnp.sum(lc_terms) / pcount

    gram = lax.dot_general(
        cmat, cmat, (((1,), (1,)), ((), ())),
        preferred_element_type=f32, precision=lax.Precision.HIGHEST)

    rows = lax.broadcasted_iota(jnp.int32, (k, k), 0)
    cols = lax.broadcasted_iota(jnp.int32, (k, k), 1)
    eye = rows == cols
    presentf = present.astype(f32)
    sq_col = jnp.sum(jnp.where(eye, sq + jnp.zeros((k, k), f32), 0.0),
                     axis=0, keepdims=True)
    present_col = jnp.sum(
        jnp.where(eye, presentf + jnp.zeros((k, k), f32), 0.0),
        axis=0, keepdims=True)

    d2 = jnp.maximum(sq + sq_col - 2.0 * gram, 0.0)
    pos = d2 > 0.0
    dist = jnp.where(pos, jnp.sqrt(jnp.where(pos, d2, 1.0)), 0.0)
    contrib = jnp.maximum(jnp.exp(MARGIN - dist) - 1.0, 0.0)
    pairmask = (presentf * present_col > 0.0) & (~eye)
    l_spread = jnp.sum(jnp.where(pairmask, contrib, 0.0))

    out_ref[0, 0] = l_center + LAMBDA_SPREAD * l_spread

  out = pl.pallas_call(
      body,
      out_shape=jax.ShapeDtypeStruct((1, 1), f32),
      in_specs=[
          pl.BlockSpec((k, dim), lambda: (0, 0)),
          pl.BlockSpec((NC, k_pad, dim), lambda: (0, 0, 0)),
          pl.BlockSpec((NC, k_pad, dim), lambda: (0, 0, 0)),
          pl.BlockSpec((NC, k_pad, dim), lambda: (0, 0, 0)),
      ],
      out_specs=pl.BlockSpec(memory_space=pltpu.SMEM),
  )(centers, s, s2, a)
  return out[0, 0]


def kernel(features, labels, centers):
  k = centers.shape[0]
  k_pad = ((k + 8 * NS - 1) // (8 * NS)) * (8 * NS)
  labels = labels.astype(jnp.int32)
  s, s2, a = _sc_segment_sums(features, labels, k_pad)
  return _tc_combine(centers, s, s2, a, k_pad)


# speedup vs baseline: 4.4063x; 1.1614x over previous
import functools

import jax
import jax.numpy as jnp
from jax import lax
from jax.experimental import pallas as pl
from jax.experimental.pallas import tpu as pltpu
from jax.experimental.pallas import tpu_sc as plsc

MARGIN = 2.5
LAMBDA_SPREAD = 0.5

NC = 2
NS = 16
L = 16


def _fill2d(ref, value):
  rows, cols = ref.shape
  vec = jnp.full((L,), value, dtype=jnp.float32)

  def body(r, carry):
    for c in range(cols // L):
      ref[r, pl.ds(c * L, L)] = vec
    return carry

  lax.fori_loop(0, rows, body, 0)


def _sc_segment_sums(features, labels, k_pad):
  batch, dim = features.shape
  nw = NC * NS
  b_per_w = batch // nw
  sub = 128
  nsub = b_per_w // sub
  stripe = k_pad // NS

  mesh = plsc.VectorSubcoreMesh(core_axis_name="c", subcore_axis_name="s")
  f32 = jnp.float32

  @functools.partial(
      pl.kernel,
      out_type=[
          jax.ShapeDtypeStruct((NC, k_pad, dim), f32),
          jax.ShapeDtypeStruct((NC, k_pad, dim), f32),
          jax.ShapeDtypeStruct((NC, k_pad, dim), f32),
      ],
      mesh=mesh,
      scratch_types=[
          pltpu.VMEM((stripe, dim), f32),
          pltpu.VMEM((sub, dim), f32),
          pltpu.VMEM((sub, dim), f32),
          pltpu.VMEM((sub, dim), f32),
          pltpu.VMEM((sub, dim), f32),
          pltpu.VMEM((sub, dim), f32),
      ] + [pltpu.VMEM((sub,), jnp.int32) for _ in range(nsub)]
        + [pltpu.SemaphoreType.DMA for _ in range(7)] + [
          pltpu.VMEM_SHARED((k_pad, dim), f32),
          pltpu.VMEM_SHARED((k_pad, dim), f32),
          pltpu.VMEM_SHARED((k_pad, dim), f32),
      ],
  )
  def sc(feats_hbm, labels_hbm, s_out, s2_out, a_out,
         zbuf, fbuf_a, fbuf_b, sqbuf_a, sqbuf_b, ones_v, *rest):
    lab_refs = rest[:nsub]
    lsem_a, lsem_b, fsem_a, fsem_b, qsem_a, qsem_b, osem = rest[nsub:nsub + 7]
    shared_s, shared_s2, shared_a = rest[nsub + 7:]
    cid = lax.axis_index("c")
    sid = lax.axis_index("s")

    fbufs = (fbuf_a, fbuf_b)
    sqbufs = (sqbuf_a, sqbuf_b)
    lsems = (lsem_a, lsem_b)
    fsems = (fsem_a, fsem_b)
    qsems = (qsem_a, qsem_b)

    base = (cid * NS + sid) * b_per_w
    for j in range(nsub):
      pltpu.sync_copy(labels_hbm.at[pl.ds(base + j * sub, sub)], lab_refs[j])

    load0 = pltpu.async_copy(
        feats_hbm.at[pl.ds(base, sub)], fbufs[0], lsems[0])

    _fill2d(zbuf, 0.0)
    _fill2d(ones_v, 1.0)

    row0 = sid * stripe
    pltpu.sync_copy(zbuf, shared_s.at[pl.ds(row0, stripe)])
    pltpu.sync_copy(zbuf, shared_s2.at[pl.ds(row0, stripe)])
    pltpu.sync_copy(zbuf, shared_a.at[pl.ds(row0, stripe)])
    plsc.subcore_barrier()

    loads = [load0] + [None] * (nsub - 1)
    scats = [None] * nsub
    ones_descs = []
    for j in range(nsub):
      b = j % 2
      if j == 1:
        loads[1] = pltpu.async_copy(
            feats_hbm.at[pl.ds(base + sub, sub)], fbufs[1], lsems[1])
      loads[j].wait()

      def sq_body(r, carry, fb=fbufs[b], qb=sqbufs[b]):
        for c in range(dim // L):
          v = fb[r, pl.ds(c * L, L)]
          qb[r, pl.ds(c * L, L)] = v * v
        return carry

      lax.fori_loop(0, sub, sq_body, 0)

      scats[j] = (
          pltpu.async_copy(fbufs[b], shared_s.at[lab_refs[j]], fsems[b],
                           add=True),
          pltpu.async_copy(sqbufs[b], shared_s2.at[lab_refs[j]], qsems[b],
                           add=True),
      )
      ones_descs.append(
          pltpu.async_copy(ones_v, shared_a.at[lab_refs[j]], osem, add=True))

      if j + 2 < nsub:
        for d in scats[j]:
          d.wait()
        loads[j + 2] = pltpu.async_copy(
            feats_hbm.at[pl.ds(base + (j + 2) * sub, sub)], fbufs[b], lsems[b])

    for j in range(max(nsub - 2, 0), nsub):
      for d in scats[j]:
        d.wait()
    for d in ones_descs:
      d.wait()

    plsc.subcore_barrier()
    pltpu.sync_copy(shared_s.at[pl.ds(row0, stripe)],
                    s_out.at[cid, pl.ds(row0, stripe)])
    pltpu.sync_copy(shared_s2.at[pl.ds(row0, stripe)],
                    s2_out.at[cid, pl.ds(row0, stripe)])
    pltpu.sync_copy(shared_a.at[pl.ds(row0, stripe)],
                    a_out.at[cid, pl.ds(row0, stripe)])

  return sc(features, labels)


def _tc_pairwise(centers):
  k, dim = centers.shape
  f32 = jnp.float32

  def body(c_ref, m_ref):
    cmat = c_ref[:]
    sq = jnp.sum(cmat * cmat, axis=1, keepdims=True)
    gram = lax.dot_general(
        cmat, cmat, (((1,), (1,)), ((), ())),
        preferred_element_type=f32, precision=lax.Precision.HIGHEST)
    rows = lax.broadcasted_iota(jnp.int32, (k, k), 0)
    cols = lax.broadcasted_iota(jnp.int32, (k, k), 1)
    eye = rows == cols
    sq_col = jnp.sum(jnp.where(eye, sq + jnp.zeros((k, k), f32), 0.0),
                     axis=0, keepdims=True)
    d2 = jnp.maximum(sq + sq_col - 2.0 * gram, 0.0)
    pos = d2 > 0.0
    dist = jnp.where(pos, jnp.sqrt(jnp.where(pos, d2, 1.0)), 0.0)
    contrib = jnp.maximum(jnp.exp(MARGIN - dist) - 1.0, 0.0)
    m_ref[...] = jnp.where(eye, 0.0, contrib)

  return pl.pallas_call(
      body,
      out_shape=jax.ShapeDtypeStruct((k, k), f32),
      in_specs=[pl.BlockSpec((k, dim), lambda: (0, 0))],
      out_specs=pl.BlockSpec((k, k), lambda: (0, 0)),
  )(centers)


def _tc_combine(centers, pair_m, s, s2, a, k_pad):
  k, dim = centers.shape
  f32 = jnp.float32

  def body(c_ref, m_ref, s_ref, s2_ref, a_ref, out_ref):
    cmat = c_ref[:]
    smat = (s_ref[0] + s_ref[1])[:k]
    s2mat = (s2_ref[0] + s2_ref[1])[:k]
    nvec = (a_ref[0] + a_ref[1])[:k, 0:1]

    q = jnp.sum(s2mat, axis=1, keepdims=True)
    cdots = jnp.sum(cmat * smat, axis=1, keepdims=True)
    sq = jnp.sum(cmat * cmat, axis=1, keepdims=True)

    present = nvec > 0.0
    pcount = jnp.sum(present.astype(f32))
    safe_n = jnp.where(present, nvec, 1.0)
    lc_terms = jnp.where(present, (q - 2.0 * cdots + nvec * sq) / safe_n, 0.0)
    l_center = jnp.sum(lc_terms) / pcount

    rows = lax.broadcasted_iota(jnp.int32, (k, k), 0)
    cols = lax.broadcasted_iota(jnp.int32, (k, k), 1)
    eye = rows == cols
    presentf = present.astype(f32)
    present_col = jnp.sum(
        jnp.where(eye, presentf + jnp.zeros((k, k), f32), 0.0),
        axis=0, keepdims=True)
    pairmask = (presentf * present_col) > 0.0
    l_spread = jnp.sum(jnp.where(pairmask, m_ref[...], 0.0))

    out_ref[0, 0] = l_center + LAMBDA_SPREAD * l_spread

  out = pl.pallas_call(
      body,
      out_shape=jax.ShapeDtypeStruct((1, 1), f32),
      in_specs=[
          pl.BlockSpec((k, dim), lambda: (0, 0)),
          pl.BlockSpec((k, k), lambda: (0, 0)),
          pl.BlockSpec((NC, k_pad, dim), lambda: (0, 0, 0)),
          pl.BlockSpec((NC, k_pad, dim), lambda: (0, 0, 0)),
          pl.BlockSpec((NC, k_pad, dim), lambda: (0, 0, 0)),
      ],
      out_specs=pl.BlockSpec(memory_space=pltpu.SMEM),
  )(centers, pair_m, s, s2, a)
  return out[0, 0]


def kernel(features, labels, centers):
  k = centers.shape[0]
  k_pad = ((k + 8 * NS - 1) // (8 * NS)) * (8 * NS)
  labels = labels.astype(jnp.int32)
  s, s2, a = _sc_segment_sums(features, labels, k_pad)
  pair_m = _tc_pairwise(centers)
  return _tc_combine(centers, pair_m, s, s2, a, k_pad)


# speedup vs baseline: 4.4594x; 1.0121x over previous
import functools

import jax
import jax.numpy as jnp
from jax import lax
from jax.experimental import pallas as pl
from jax.experimental.pallas import tpu as pltpu
from jax.experimental.pallas import tpu_sc as plsc

MARGIN = 2.5
LAMBDA_SPREAD = 0.5

NC = 2
NS = 16
L = 16
CW = 128


def _fill2d(ref, value):
  rows, cols = ref.shape
  vec = jnp.full((L,), value, dtype=jnp.float32)

  def body(r, carry):
    for c in range(cols // L):
      ref[r, pl.ds(c * L, L)] = vec
    return carry

  lax.fori_loop(0, rows, body, 0)


def _sc_segment_sums(features, labels, k_pad):
  batch, dim = features.shape
  nw = NC * NS
  b_per_w = batch // nw
  sub = 128
  nsub = b_per_w // sub
  stripe = k_pad // NS

  mesh = plsc.VectorSubcoreMesh(core_axis_name="c", subcore_axis_name="s")
  f32 = jnp.float32

  @functools.partial(
      pl.kernel,
      out_type=[
          jax.ShapeDtypeStruct((NC, k_pad, dim), f32),
          jax.ShapeDtypeStruct((NC, k_pad, dim), f32),
          jax.ShapeDtypeStruct((NC, k_pad, CW), f32),
      ],
      mesh=mesh,
      scratch_types=[
          pltpu.VMEM((stripe, dim), f32),
          pltpu.VMEM((sub, dim), f32),
          pltpu.VMEM((sub, dim), f32),
          pltpu.VMEM((sub, dim), f32),
          pltpu.VMEM((sub, dim), f32),
          pltpu.VMEM((sub, dim), f32),
      ] + [pltpu.VMEM((sub,), jnp.int32) for _ in range(nsub)]
        + [pltpu.SemaphoreType.DMA for _ in range(7)] + [
          pltpu.VMEM_SHARED((k_pad, dim), f32),
          pltpu.VMEM_SHARED((k_pad, dim), f32),
          pltpu.VMEM_SHARED((k_pad, dim), f32),
      ],
  )
  def sc(feats_hbm, labels_hbm, s_out, s2_out, a_out,
         zbuf, fbuf_a, fbuf_b, sqbuf_a, sqbuf_b, ones_v, *rest):
    lab_refs = rest[:nsub]
    lsem_a, lsem_b, fsem_a, fsem_b, qsem_a, qsem_b, osem = rest[nsub:nsub + 7]
    shared_s, shared_s2, shared_a = rest[nsub + 7:]
    cid = lax.axis_index("c")
    sid = lax.axis_index("s")

    fbufs = (fbuf_a, fbuf_b)
    sqbufs = (sqbuf_a, sqbuf_b)
    lsems = (lsem_a, lsem_b)
    fsems = (fsem_a, fsem_b)
    qsems = (qsem_a, qsem_b)

    base = (cid * NS + sid) * b_per_w
    for j in range(nsub):
      pltpu.sync_copy(labels_hbm.at[pl.ds(base + j * sub, sub)], lab_refs[j])

    load0 = pltpu.async_copy(
        feats_hbm.at[pl.ds(base, sub)], fbufs[0], lsems[0])

    _fill2d(zbuf, 0.0)
    _fill2d(ones_v, 1.0)

    row0 = sid * stripe
    pltpu.sync_copy(zbuf, shared_s.at[pl.ds(row0, stripe)])
    pltpu.sync_copy(zbuf, shared_s2.at[pl.ds(row0, stripe)])
    pltpu.sync_copy(zbuf, shared_a.at[pl.ds(row0, stripe)])
    plsc.subcore_barrier()

    loads = [load0] + [None] * (nsub - 1)
    scats = [None] * nsub
    ones_descs = []
    for j in range(nsub):
      b = j % 2
      if j == 1:
        loads[1] = pltpu.async_copy(
            feats_hbm.at[pl.ds(base + sub, sub)], fbufs[1], lsems[1])
      loads[j].wait()

      def sq_body(r, carry, fb=fbufs[b], qb=sqbufs[b]):
        for c in range(dim // L):
          v = fb[r, pl.ds(c * L, L)]
          qb[r, pl.ds(c * L, L)] = v * v
        return carry

      lax.fori_loop(0, sub, sq_body, 0)

      scats[j] = (
          pltpu.async_copy(fbufs[b], shared_s.at[lab_refs[j]], fsems[b],
                           add=True),
          pltpu.async_copy(sqbufs[b], shared_s2.at[lab_refs[j]], qsems[b],
                           add=True),
      )
      ones_descs.append(
          pltpu.async_copy(ones_v, shared_a.at[lab_refs[j]], osem, add=True))

      if j + 2 < nsub:
        for d in scats[j]:
          d.wait()
        loads[j + 2] = pltpu.async_copy(
            feats_hbm.at[pl.ds(base + (j + 2) * sub, sub)], fbufs[b], lsems[b])

    for j in range(max(nsub - 2, 0), nsub):
      for d in scats[j]:
        d.wait()
    for d in ones_descs:
      d.wait()

    plsc.subcore_barrier()
    pltpu.sync_copy(shared_s.at[pl.ds(row0, stripe)],
                    s_out.at[cid, pl.ds(row0, stripe)])
    pltpu.sync_copy(shared_s2.at[pl.ds(row0, stripe)],
                    s2_out.at[cid, pl.ds(row0, stripe)])
    pltpu.sync_copy(shared_a.at[pl.ds(row0, stripe)],
                    a_out.at[cid, pl.ds(row0, stripe)])

  return sc(features, labels)


def _tc_pairwise(centers):
  k, dim = centers.shape
  f32 = jnp.float32

  def body(c_ref, m_ref):
    cmat = c_ref[:]
    sq = jnp.sum(cmat * cmat, axis=1, keepdims=True)
    gram = lax.dot_general(
        cmat, cmat, (((1,), (1,)), ((), ())),
        preferred_element_type=f32, precision=lax.Precision.HIGHEST)
    rows = lax.broadcasted_iota(jnp.int32, (k, k), 0)
    cols = lax.broadcasted_iota(jnp.int32, (k, k), 1)
    eye = rows == cols
    sq_col = jnp.sum(jnp.where(eye, sq + jnp.zeros((k, k), f32), 0.0),
                     axis=0, keepdims=True)
    d2 = jnp.maximum(sq + sq_col - 2.0 * gram, 0.0)
    pos = d2 > 0.0
    dist = jnp.where(pos, jnp.sqrt(jnp.where(pos, d2, 1.0)), 0.0)
    contrib = jnp.maximum(jnp.exp(MARGIN - dist) - 1.0, 0.0)
    m_ref[...] = jnp.where(eye, 0.0, contrib).astype(jnp.bfloat16)

  return pl.pallas_call(
      body,
      out_shape=jax.ShapeDtypeStruct((k, k), jnp.bfloat16),
      in_specs=[pl.BlockSpec((k, dim), lambda: (0, 0))],
      out_specs=pl.BlockSpec((k, k), lambda: (0, 0)),
  )(centers)


def _tc_combine(centers, pair_m, s, s2, a, k_pad):
  k, dim = centers.shape
  f32 = jnp.float32

  def body(c_ref, m_ref, s_ref, s2_ref, a_ref, out_ref):
    cmat = c_ref[:]
    smat = (s_ref[0] + s_ref[1])[:k]
    s2mat = (s2_ref[0] + s2_ref[1])[:k]
    nvec = (a_ref[0] + a_ref[1])[:k, 0:1]

    q = jnp.sum(s2mat, axis=1, keepdims=True)
    cdots = jnp.sum(cmat * smat, axis=1, keepdims=True)
    sq = jnp.sum(cmat * cmat, axis=1, keepdims=True)

    present = nvec > 0.0
    pcount = jnp.sum(present.astype(f32))
    safe_n = jnp.where(present, nvec, 1.0)
    lc_terms = jnp.where(present, (q - 2.0 * cdots + nvec * sq) / safe_n, 0.0)
    l_center = jnp.sum(lc_terms) / pcount

    rows = lax.broadcasted_iota(jnp.int32, (k, k), 0)
    cols = lax.broadcasted_iota(jnp.int32, (k, k), 1)
    eye = rows == cols
    presentf = present.astype(f32)
    present_col = jnp.sum(
        jnp.where(eye, presentf + jnp.zeros((k, k), f32), 0.0),
        axis=0, keepdims=True)
    pairmask = (presentf * present_col) > 0.0
    mvals = m_ref[...].astype(f32)
    l_spread = jnp.sum(jnp.where(pairmask, mvals, 0.0))

    out_ref[0, 0] = l_center + LAMBDA_SPREAD * l_spread

  out = pl.pallas_call(
      body,
      out_shape=jax.ShapeDtypeStruct((1, 1), f32),
      in_specs=[
          pl.BlockSpec((k, dim), lambda: (0, 0)),
          pl.BlockSpec((k, k), lambda: (0, 0)),
          pl.BlockSpec((NC, k_pad, dim), lambda: (0, 0, 0)),
          pl.BlockSpec((NC, k_pad, dim), lambda: (0, 0, 0)),
          pl.BlockSpec((NC, k_pad, CW), lambda: (0, 0, 0)),
      ],
      out_specs=pl.BlockSpec(memory_space=pltpu.SMEM),
  )(centers, pair_m, s, s2, a)
  return out[0, 0]


def kernel(features, labels, centers):
  k = centers.shape[0]
  k_pad = ((k + 8 * NS - 1) // (8 * NS)) * (8 * NS)
  labels = labels.astype(jnp.int32)
  s, s2, a = _sc_segment_sums(features, labels, k_pad)
  pair_m = _tc_pairwise(centers)
  return _tc_combine(centers, pair_m, s, s2, a, k_pad)


# speedup vs baseline: 4.8842x; 1.0953x over previous
import functools

import jax
import jax.numpy as jnp
from jax import lax
from jax.experimental import pallas as pl
from jax.experimental.pallas import tpu as pltpu
from jax.experimental.pallas import tpu_sc as plsc

MARGIN = 2.5
LAMBDA_SPREAD = 0.5

NC = 2
NS = 16
L = 16


def _fill2d(ref, value):
  rows, cols = ref.shape
  vec = jnp.full((L,), value, dtype=jnp.float32)

  def body(r, carry):
    for c in range(cols // L):
      ref[r, pl.ds(c * L, L)] = vec
    return carry

  lax.fori_loop(0, rows, body, 0)


def _sc_segment_sums(features, labels, k_pad):
  batch, dim = features.shape
  nw = NC * NS
  b_per_w = batch // nw
  sub = 128
  nsub = b_per_w // sub
  stripe = k_pad // NS

  mesh = plsc.VectorSubcoreMesh(core_axis_name="c", subcore_axis_name="s")
  f32 = jnp.float32

  @functools.partial(
      pl.kernel,
      out_type=[
          jax.ShapeDtypeStruct((NC, k_pad, dim), f32),
          jax.ShapeDtypeStruct((NC, k_pad, dim), f32),
      ],
      mesh=mesh,
      scratch_types=[
          pltpu.VMEM((stripe, dim), f32),
          pltpu.VMEM((sub, dim), f32),
          pltpu.VMEM((sub, dim), f32),
          pltpu.VMEM((sub, dim), f32),
          pltpu.VMEM((sub, dim), f32),
      ] + [pltpu.VMEM((sub,), jnp.int32) for _ in range(nsub)]
        + [pltpu.SemaphoreType.DMA for _ in range(6)] + [
          pltpu.VMEM_SHARED((k_pad, dim), f32),
          pltpu.VMEM_SHARED((k_pad, dim), f32),
      ],
  )
  def sc(feats_hbm, labels_hbm, s_out, s2_out,
         zbuf, fbuf_a, fbuf_b, sqbuf_a, sqbuf_b, *rest):
    lab_refs = rest[:nsub]
    lsem_a, lsem_b, fsem_a, fsem_b, qsem_a, qsem_b = rest[nsub:nsub + 6]
    shared_s, shared_s2 = rest[nsub + 6:]
    cid = lax.axis_index("c")
    sid = lax.axis_index("s")

    fbufs = (fbuf_a, fbuf_b)
    sqbufs = (sqbuf_a, sqbuf_b)
    lsems = (lsem_a, lsem_b)
    fsems = (fsem_a, fsem_b)
    qsems = (qsem_a, qsem_b)

    base = (cid * NS + sid) * b_per_w
    for j in range(nsub):
      pltpu.sync_copy(labels_hbm.at[pl.ds(base + j * sub, sub)], lab_refs[j])

    load0 = pltpu.async_copy(
        feats_hbm.at[pl.ds(base, sub)], fbufs[0], lsems[0])

    _fill2d(zbuf, 0.0)

    row0 = sid * stripe
    pltpu.sync_copy(zbuf, shared_s.at[pl.ds(row0, stripe)])
    pltpu.sync_copy(zbuf, shared_s2.at[pl.ds(row0, stripe)])
    plsc.subcore_barrier()

    lane = lax.iota(jnp.int32, L)
    shift_idx = (lane + (L - 1)) & (L - 1)

    loads = [load0] + [None] * (nsub - 1)
    scats = [None] * nsub
    for j in range(nsub):
      b = j % 2
      if j == 1:
        loads[1] = pltpu.async_copy(
            feats_hbm.at[pl.ds(base + sub, sub)], fbufs[1], lsems[1])
      loads[j].wait()

      def sq_body(r, carry, fb=fbufs[b], qb=sqbufs[b]):
        v0 = fb[r, pl.ds(0, L)]
        sq0 = v0 * v0
        sh = sq0.at[shift_idx].get(mode="promise_in_bounds")
        merged = jnp.where(lane == 1, sq0 + sh, sq0)
        qb[r, pl.ds(0, L)] = jnp.where(lane == 0, 1.0, merged)
        for c in range(1, dim // L):
          v = fb[r, pl.ds(c * L, L)]
          qb[r, pl.ds(c * L, L)] = v * v
        return carry

      lax.fori_loop(0, sub, sq_body, 0)

      scats[j] = (
          pltpu.async_copy(fbufs[b], shared_s.at[lab_refs[j]], fsems[b],
                           add=True),
          pltpu.async_copy(sqbufs[b], shared_s2.at[lab_refs[j]], qsems[b],
                           add=True),
      )

      if j + 2 < nsub:
        for d in scats[j]:
          d.wait()
        loads[j + 2] = pltpu.async_copy(
            feats_hbm.at[pl.ds(base + (j + 2) * sub, sub)], fbufs[b], lsems[b])

    for j in range(max(nsub - 2, 0), nsub):
      for d in scats[j]:
        d.wait()

    plsc.subcore_barrier()
    pltpu.sync_copy(shared_s.at[pl.ds(row0, stripe)],
                    s_out.at[cid, pl.ds(row0, stripe)])
    pltpu.sync_copy(shared_s2.at[pl.ds(row0, stripe)],
                    s2_out.at[cid, pl.ds(row0, stripe)])

  return sc(features, labels)


def _tc_pairwise(centers):
  k, dim = centers.shape
  f32 = jnp.float32

  def body(c_ref, m_ref):
    cmat = c_ref[:]
    sq = jnp.sum(cmat * cmat, axis=1, keepdims=True)
    gram = lax.dot_general(
        cmat, cmat, (((1,), (1,)), ((), ())),
        preferred_element_type=f32, precision=lax.Precision.HIGHEST)
    rows = lax.broadcasted_iota(jnp.int32, (k, k), 0)
    cols = lax.broadcasted_iota(jnp.int32, (k, k), 1)
    eye = rows == cols
    sq_col = jnp.sum(jnp.where(eye, sq + jnp.zeros((k, k), f32), 0.0),
                     axis=0, keepdims=True)
    d2 = jnp.maximum(sq + sq_col - 2.0 * gram, 0.0)
    pos = d2 > 0.0
    dist = jnp.where(pos, jnp.sqrt(jnp.where(pos, d2, 1.0)), 0.0)
    contrib = jnp.maximum(jnp.exp(MARGIN - dist) - 1.0, 0.0)
    m_ref[...] = jnp.where(eye, 0.0, contrib).astype(jnp.bfloat16)

  return pl.pallas_call(
      body,
      out_shape=jax.ShapeDtypeStruct((k, k), jnp.bfloat16),
      in_specs=[pl.BlockSpec((k, dim), lambda: (0, 0))],
      out_specs=pl.BlockSpec((k, k), lambda: (0, 0)),
  )(centers)


def _tc_combine(centers, pair_m, s, s2, k_pad):
  k, dim = centers.shape
  f32 = jnp.float32

  def body(c_ref, m_ref, s_ref, s2_ref, out_ref):
    cmat = c_ref[:]
    smat = (s_ref[0] + s_ref[1])[:k]
    s2mat = (s2_ref[0] + s2_ref[1])[:k]
    nvec = s2mat[:, 0:1]

    q = jnp.sum(s2mat, axis=1, keepdims=True) - nvec
    cdots = jnp.sum(cmat * smat, axis=1, keepdims=True)
    sq = jnp.sum(cmat * cmat, axis=1, keepdims=True)

    present = nvec > 0.0
    pcount = jnp.sum(present.astype(f32))
    safe_n = jnp.where(present, nvec, 1.0)
    lc_terms = jnp.where(present, (q - 2.0 * cdots + nvec * sq) / safe_n, 0.0)
    l_center = jnp.sum(lc_terms) / pcount

    rows = lax.broadcasted_iota(jnp.int32, (k, k), 0)
    cols = lax.broadcasted_iota(jnp.int32, (k, k), 1)
    eye = rows == cols
    presentf = present.astype(f32)
    present_col = jnp.sum(
        jnp.where(eye, presentf + jnp.zeros((k, k), f32), 0.0),
        axis=0, keepdims=True)
    pairmask = (presentf * present_col) > 0.0
    mvals = m_ref[...].astype(f32)
    l_spread = jnp.sum(jnp.where(pairmask, mvals, 0.0))

    out_ref[0, 0] = l_center + LAMBDA_SPREAD * l_spread

  out = pl.pallas_call(
      body,
      out_shape=jax.ShapeDtypeStruct((1, 1), f32),
      in_specs=[
          pl.BlockSpec((k, dim), lambda: (0, 0)),
          pl.BlockSpec((k, k), lambda: (0, 0)),
          pl.BlockSpec((NC, k_pad, dim), lambda: (0, 0, 0)),
          pl.BlockSpec((NC, k_pad, dim), lambda: (0, 0, 0)),
      ],
      out_specs=pl.BlockSpec(memory_space=pltpu.SMEM),
  )(centers, pair_m, s, s2)
  return out[0, 0]


def kernel(features, labels, centers):
  k = centers.shape[0]
  k_pad = ((k + 8 * NS - 1) // (8 * NS)) * (8 * NS)
  labels = labels.astype(jnp.int32)
  s, s2 = _sc_segment_sums(features, labels, k_pad)
  pair_m = _tc_pairwise(centers)
  return _tc_combine(centers, pair_m, s, s2, k_pad)


# speedup vs baseline: 4.9807x; 1.0198x over previous
import functools

import jax
import jax.numpy as jnp
from jax import lax
from jax.experimental import pallas as pl
from jax.experimental.pallas import tpu as pltpu
from jax.experimental.pallas import tpu_sc as plsc

MARGIN = 2.5
LAMBDA_SPREAD = 0.5

NC = 2
NS = 16
L = 16


def _fill2d(ref, value):
  rows, cols = ref.shape
  vec = jnp.full((L,), value, dtype=jnp.float32)

  def body(r, carry):
    for c in range(cols // L):
      ref[r, pl.ds(c * L, L)] = vec
    return carry

  lax.fori_loop(0, rows, body, 0)


def _sc_segment_sums(features, labels, k_pad):
  batch, dim = features.shape
  nw = NC * NS
  b_per_w = batch // nw
  sub = 128
  nsub = b_per_w // sub
  stripe = k_pad // NS

  mesh = plsc.VectorSubcoreMesh(core_axis_name="c", subcore_axis_name="s")
  f32 = jnp.float32

  @functools.partial(
      pl.kernel,
      out_type=[
          jax.ShapeDtypeStruct((NC, k_pad, dim), f32),
          jax.ShapeDtypeStruct((NC, k_pad, dim), f32),
      ],
      mesh=mesh,
      scratch_types=[
          pltpu.VMEM((stripe, dim), f32),
          pltpu.VMEM((sub, dim), f32),
          pltpu.VMEM((sub, dim), f32),
          pltpu.VMEM((sub, dim), f32),
          pltpu.VMEM((sub, dim), f32),
      ] + [pltpu.VMEM((sub,), jnp.int32) for _ in range(nsub)]
        + [pltpu.SemaphoreType.DMA for _ in range(6)] + [
          pltpu.VMEM_SHARED((k_pad, dim), f32),
          pltpu.VMEM_SHARED((k_pad, dim), f32),
      ],
  )
  def sc(feats_hbm, labels_hbm, s_out, s2_out,
         zbuf, fbuf_a, fbuf_b, sqbuf_a, sqbuf_b, *rest):
    lab_refs = rest[:nsub]
    lsem_a, lsem_b, fsem_a, fsem_b, qsem_a, qsem_b = rest[nsub:nsub + 6]
    shared_s, shared_s2 = rest[nsub + 6:]
    cid = lax.axis_index("c")
    sid = lax.axis_index("s")

    fbufs = (fbuf_a, fbuf_b)
    sqbufs = (sqbuf_a, sqbuf_b)
    lsems = (lsem_a, lsem_b)
    fsems = (fsem_a, fsem_b)
    qsems = (qsem_a, qsem_b)

    base = (cid * NS + sid) * b_per_w
    for j in range(nsub):
      pltpu.sync_copy(labels_hbm.at[pl.ds(base + j * sub, sub)], lab_refs[j])

    load0 = pltpu.async_copy(
        feats_hbm.at[pl.ds(base, sub)], fbufs[0], lsems[0])

    _fill2d(zbuf, 0.0)

    row0 = sid * stripe
    pltpu.sync_copy(zbuf, shared_s.at[pl.ds(row0, stripe)])
    pltpu.sync_copy(zbuf, shared_s2.at[pl.ds(row0, stripe)])
    plsc.subcore_barrier()

    lane = lax.iota(jnp.int32, L)
    shift_idx = (lane + (L - 1)) & (L - 1)

    loads = [load0] + [None] * (nsub - 1)
    scats = [None] * nsub
    for j in range(nsub):
      b = j % 2
      if j == 1:
        loads[1] = pltpu.async_copy(
            feats_hbm.at[pl.ds(base + sub, sub)], fbufs[1], lsems[1])
      loads[j].wait()

      def sq_body(r, carry, fb=fbufs[b], qb=sqbufs[b]):
        v0 = fb[r, pl.ds(0, L)]
        sq0 = v0 * v0
        sh = sq0.at[shift_idx].get(mode="promise_in_bounds")
        merged = jnp.where(lane == 1, sq0 + sh, sq0)
        qb[r, pl.ds(0, L)] = jnp.where(lane == 0, 1.0, merged)
        for c in range(1, dim // L):
          v = fb[r, pl.ds(c * L, L)]
          qb[r, pl.ds(c * L, L)] = v * v
        return carry

      lax.fori_loop(0, sub, sq_body, 0)

      scats[j] = (
          pltpu.async_copy(fbufs[b], shared_s.at[lab_refs[j]], fsems[b],
                           add=True),
          pltpu.async_copy(sqbufs[b], shared_s2.at[lab_refs[j]], qsems[b],
                           add=True),
      )

      if j + 2 < nsub:
        for d in scats[j]:
          d.wait()
        loads[j + 2] = pltpu.async_copy(
            feats_hbm.at[pl.ds(base + (j + 2) * sub, sub)], fbufs[b], lsems[b])

    for j in range(max(nsub - 2, 0), nsub):
      for d in scats[j]:
        d.wait()

    plsc.subcore_barrier()
    pltpu.sync_copy(shared_s.at[pl.ds(row0, stripe)],
                    s_out.at[cid, pl.ds(row0, stripe)])
    pltpu.sync_copy(shared_s2.at[pl.ds(row0, stripe)],
                    s2_out.at[cid, pl.ds(row0, stripe)])

  return sc(features, labels)


def _tc_pairwise(centers):
  k, dim = centers.shape
  f32 = jnp.float32

  def body(c_ref, m_ref):
    cmat = c_ref[:]
    sq = jnp.sum(cmat * cmat, axis=1, keepdims=True)
    gram = lax.dot_general(
        cmat, cmat, (((1,), (1,)), ((), ())),
        preferred_element_type=f32, precision=lax.Precision.HIGHEST)
    rows = lax.broadcasted_iota(jnp.int32, (k, k), 0)
    cols = lax.broadcasted_iota(jnp.int32, (k, k), 1)
    eye = rows == cols
    sq_col = jnp.sum(jnp.where(eye, sq + jnp.zeros((k, k), f32), 0.0),
                     axis=0, keepdims=True)
    d2 = jnp.maximum(sq + sq_col - 2.0 * gram, 0.0)
    pos = d2 > 0.0
    dist = jnp.where(pos, jnp.sqrt(jnp.where(pos, d2, 1.0)), 0.0)
    contrib = jnp.maximum(jnp.exp(MARGIN - dist) - 1.0, 0.0)
    m_ref[...] = jnp.where(eye, 0.0, contrib).astype(jnp.bfloat16)

  return pl.pallas_call(
      body,
      out_shape=jax.ShapeDtypeStruct((k, k), jnp.bfloat16),
      in_specs=[pl.BlockSpec((k, dim), lambda: (0, 0))],
      out_specs=pl.BlockSpec((k, k), lambda: (0, 0)),
  )(centers)


def _tc_combine(centers, pair_m, s, s2, k_pad):
  k, dim = centers.shape
  f32 = jnp.float32

  def body(c_ref, m_ref, s_ref, s2_ref, out_ref):
    cmat = c_ref[:]
    smat = (s_ref[0] + s_ref[1])[:k]
    s2mat = (s2_ref[0] + s2_ref[1])[:k]
    nvec = s2mat[:, 0:1]

    q = jnp.sum(s2mat, axis=1, keepdims=True) - nvec
    cdots = jnp.sum(cmat * smat, axis=1, keepdims=True)
    sq = jnp.sum(cmat * cmat, axis=1, keepdims=True)

    present = nvec > 0.0
    pcount = jnp.sum(present.astype(f32))
    safe_n = jnp.where(present, nvec, 1.0)
    lc_terms = jnp.where(present, (q - 2.0 * cdots + nvec * sq) / safe_n, 0.0)
    l_center = jnp.sum(lc_terms) / pcount

    presentf = present.astype(f32)
    p_bf = presentf.astype(jnp.bfloat16)
    mp = lax.dot_general(
        m_ref[...], p_bf, (((1,), (0,)), ((), ())),
        preferred_element_type=f32)
    l_spread = jnp.sum(presentf * mp)

    out_ref[0, 0] = l_center + LAMBDA_SPREAD * l_spread

  out = pl.pallas_call(
      body,
      out_shape=jax.ShapeDtypeStruct((1, 1), f32),
      in_specs=[
          pl.BlockSpec((k, dim), lambda: (0, 0)),
          pl.BlockSpec((k, k), lambda: (0, 0)),
          pl.BlockSpec((NC, k_pad, dim), lambda: (0, 0, 0)),
          pl.BlockSpec((NC, k_pad, dim), lambda: (0, 0, 0)),
      ],
      out_specs=pl.BlockSpec(memory_space=pltpu.SMEM),
  )(centers, pair_m, s, s2)
  return out[0, 0]


def kernel(features, labels, centers):
  k = centers.shape[0]
  k_pad = ((k + 8 * NS - 1) // (8 * NS)) * (8 * NS)
  labels = labels.astype(jnp.int32)
  s, s2 = _sc_segment_sums(features, labels, k_pad)
  pair_m = _tc_pairwise(centers)
  return _tc_combine(centers, pair_m, s, s2, k_pad)


# speedup vs baseline: 5.0886x; 1.0217x over previous
import functools

import jax
import jax.numpy as jnp
from jax import lax
from jax.experimental import pallas as pl
from jax.experimental.pallas import tpu as pltpu
from jax.experimental.pallas import tpu_sc as plsc

MARGIN = 2.5
LAMBDA_SPREAD = 0.5

NC = 2
NS = 16
L = 16


def _fill2d(ref, value):
  rows, cols = ref.shape
  vec = jnp.full((L,), value, dtype=jnp.float32)

  def body(r, carry):
    for c in range(cols // L):
      ref[r, pl.ds(c * L, L)] = vec
    return carry

  lax.fori_loop(0, rows, body, 0)


def _sc_segment_sums(features, labels, k_pad):
  batch, dim = features.shape
  nw = NC * NS
  b_per_w = batch // nw
  sub = 128
  nsub = b_per_w // sub
  stripe = k_pad // NS

  mesh = plsc.VectorSubcoreMesh(core_axis_name="c", subcore_axis_name="s")
  f32 = jnp.float32

  @functools.partial(
      pl.kernel,
      out_type=[
          jax.ShapeDtypeStruct((NC, k_pad, dim), f32),
          jax.ShapeDtypeStruct((NC, k_pad, dim), f32),
      ],
      mesh=mesh,
      scratch_types=[
          pltpu.VMEM((stripe, dim), f32),
      ] + [pltpu.VMEM((sub, dim), f32) for _ in range(3)]
        + [pltpu.VMEM((sub, dim), f32) for _ in range(3)]
        + [pltpu.VMEM((sub,), jnp.int32) for _ in range(nsub)]
        + [pltpu.SemaphoreType.DMA for _ in range(9)] + [
          pltpu.VMEM_SHARED((k_pad, dim), f32),
          pltpu.VMEM_SHARED((k_pad, dim), f32),
      ],
  )
  def sc(feats_hbm, labels_hbm, s_out, s2_out, zbuf, *rest):
    fbufs = rest[:3]
    sqbufs = rest[3:6]
    lab_refs = rest[6:6 + nsub]
    sems = rest[6 + nsub:6 + nsub + 9]
    lsems, fsems, qsems = sems[0:3], sems[3:6], sems[6:9]
    shared_s, shared_s2 = rest[6 + nsub + 9:]
    cid = lax.axis_index("c")
    sid = lax.axis_index("s")

    base = (cid * NS + sid) * b_per_w
    for j in range(nsub):
      pltpu.sync_copy(labels_hbm.at[pl.ds(base + j * sub, sub)], lab_refs[j])

    loads = [None] * nsub
    for j in range(min(3, nsub)):
      loads[j] = pltpu.async_copy(
          feats_hbm.at[pl.ds(base + j * sub, sub)], fbufs[j], lsems[j])

    _fill2d(zbuf, 0.0)

    row0 = sid * stripe
    pltpu.sync_copy(zbuf, shared_s.at[pl.ds(row0, stripe)])
    pltpu.sync_copy(zbuf, shared_s2.at[pl.ds(row0, stripe)])
    plsc.subcore_barrier()

    lane = lax.iota(jnp.int32, L)
    shift_idx = (lane + (L - 1)) & (L - 1)

    scats = [None] * nsub
    for j in range(nsub):
      b = j % 3
      loads[j].wait()

      def sq_body(r, carry, fb=fbufs[b], qb=sqbufs[b]):
        v0 = fb[r, pl.ds(0, L)]
        sq0 = v0 * v0
        sh = sq0.at[shift_idx].get(mode="promise_in_bounds")
        merged = jnp.where(lane == 1, sq0 + sh, sq0)
        qb[r, pl.ds(0, L)] = jnp.where(lane == 0, 1.0, merged)
        for c in range(1, dim // L):
          v = fb[r, pl.ds(c * L, L)]
          qb[r, pl.ds(c * L, L)] = v * v
        return carry

      lax.fori_loop(0, sub, sq_body, 0)

      scats[j] = (
          pltpu.async_copy(fbufs[b], shared_s.at[lab_refs[j]], fsems[b],
                           add=True),
          pltpu.async_copy(sqbufs[b], shared_s2.at[lab_refs[j]], qsems[b],
                           add=True),
      )

      if j + 3 < nsub:
        for d in scats[j]:
          d.wait()
        loads[j + 3] = pltpu.async_copy(
            feats_hbm.at[pl.ds(base + (j + 3) * sub, sub)], fbufs[b], lsems[b])

    for j in range(max(nsub - 3, 0), nsub):
      for d in scats[j]:
        d.wait()

    plsc.subcore_barrier()
    pltpu.sync_copy(shared_s.at[pl.ds(row0, stripe)],
                    s_out.at[cid, pl.ds(row0, stripe)])
    pltpu.sync_copy(shared_s2.at[pl.ds(row0, stripe)],
                    s2_out.at[cid, pl.ds(row0, stripe)])

  return sc(features, labels)


def _tc_pairwise(centers):
  k, dim = centers.shape
  f32 = jnp.float32

  def body(c_ref, m_ref):
    cmat = c_ref[:]
    sq = jnp.sum(cmat * cmat, axis=1, keepdims=True)
    gram = lax.dot_general(
        cmat, cmat, (((1,), (1,)), ((), ())),
        preferred_element_type=f32, precision=lax.Precision.HIGHEST)
    rows = lax.broadcasted_iota(jnp.int32, (k, k), 0)
    cols = lax.broadcasted_iota(jnp.int32, (k, k), 1)
    eye = rows == cols
    sq_col = jnp.sum(jnp.where(eye, sq + jnp.zeros((k, k), f32), 0.0),
                     axis=0, keepdims=True)
    d2 = jnp.maximum(sq + sq_col - 2.0 * gram, 0.0)
    pos = d2 > 0.0
    dist = jnp.where(pos, jnp.sqrt(jnp.where(pos, d2, 1.0)), 0.0)
    contrib = jnp.maximum(jnp.exp(MARGIN - dist) - 1.0, 0.0)
    m_ref[...] = jnp.where(eye, 0.0, contrib).astype(jnp.bfloat16)

  return pl.pallas_call(
      body,
      out_shape=jax.ShapeDtypeStruct((k, k), jnp.bfloat16),
      in_specs=[pl.BlockSpec((k, dim), lambda: (0, 0))],
      out_specs=pl.BlockSpec((k, k), lambda: (0, 0)),
  )(centers)


def _tc_combine(centers, pair_m, s, s2, k_pad):
  k, dim = centers.shape
  f32 = jnp.float32

  def body(c_ref, m_ref, s_ref, s2_ref, out_ref):
    cmat = c_ref[:]
    smat = (s_ref[0] + s_ref[1])[:k]
    s2mat = (s2_ref[0] + s2_ref[1])[:k]
    nvec = s2mat[:, 0:1]

    q = jnp.sum(s2mat, axis=1, keepdims=True) - nvec
    cdots = jnp.sum(cmat * smat, axis=1, keepdims=True)
    sq = jnp.sum(cmat * cmat, axis=1, keepdims=True)

    present = nvec > 0.0
    pcount = jnp.sum(present.astype(f32))
    safe_n = jnp.where(present, nvec, 1.0)
    lc_terms = jnp.where(present, (q - 2.0 * cdots + nvec * sq) / safe_n, 0.0)
    l_center = jnp.sum(lc_terms) / pcount

    presentf = present.astype(f32)
    p_bf = presentf.astype(jnp.bfloat16)
    mp = lax.dot_general(
        m_ref[...], p_bf, (((1,), (0,)), ((), ())),
        preferred_element_type=f32)
    l_spread = jnp.sum(presentf * mp)

    out_ref[0, 0] = l_center + LAMBDA_SPREAD * l_spread

  out = pl.pallas_call(
      body,
      out_shape=jax.ShapeDtypeStruct((1, 1), f32),
      in_specs=[
          pl.BlockSpec((k, dim), lambda: (0, 0)),
          pl.BlockSpec((k, k), lambda: (0, 0)),
          pl.BlockSpec((NC, k_pad, dim), lambda: (0, 0, 0)),
          pl.BlockSpec((NC, k_pad, dim), lambda: (0, 0, 0)),
      ],
      out_specs=pl.BlockSpec(memory_space=pltpu.SMEM),
  )(centers, pair_m, s, s2)
  return out[0, 0]


def kernel(features, labels, centers):
  k = centers.shape[0]
  k_pad = ((k + 8 * NS - 1) // (8 * NS)) * (8 * NS)
  labels = labels.astype(jnp.int32)
  s, s2 = _sc_segment_sums(features, labels, k_pad)
  pair_m = _tc_pairwise(centers)
  return _tc_combine(centers, pair_m, s, s2, k_pad)


# speedup vs baseline: 5.0991x; 1.0021x over previous
import functools

import jax
import jax.numpy as jnp
from jax import lax
from jax.experimental import pallas as pl
from jax.experimental.pallas import tpu as pltpu
from jax.experimental.pallas import tpu_sc as plsc

MARGIN = 2.5
LAMBDA_SPREAD = 0.5

NC = 2
NS = 16
L = 16


def _fill2d(ref, value):
  rows, cols = ref.shape
  vec = jnp.full((L,), value, dtype=jnp.float32)

  def body(r, carry):
    for c in range(cols // L):
      ref[r, pl.ds(c * L, L)] = vec
    return carry

  lax.fori_loop(0, rows, body, 0)


def _sc_segment_sums(features, labels, k_pad):
  batch, dim = features.shape
  nw = NC * NS
  b_per_w = batch // nw
  sub = 128
  nsub = b_per_w // sub
  stripe = k_pad // NS

  mesh = plsc.VectorSubcoreMesh(core_axis_name="c", subcore_axis_name="s")
  f32 = jnp.float32

  @functools.partial(
      pl.kernel,
      out_type=[
          jax.ShapeDtypeStruct((NC, k_pad, dim), f32),
          jax.ShapeDtypeStruct((NC, k_pad, dim), f32),
      ],
      mesh=mesh,
      scratch_types=[
          pltpu.VMEM((stripe, dim), f32),
      ] + [pltpu.VMEM((sub, dim), f32) for _ in range(3)]
        + [pltpu.VMEM((sub, dim), f32) for _ in range(3)]
        + [pltpu.VMEM((sub,), jnp.int32) for _ in range(nsub)]
        + [pltpu.SemaphoreType.DMA for _ in range(9)] + [
          pltpu.VMEM_SHARED((k_pad, dim), f32),
          pltpu.VMEM_SHARED((k_pad, dim), f32),
      ],
  )
  def sc(feats_hbm, labels_hbm, s_out, s2_out, zbuf, *rest):
    fbufs = rest[:3]
    sqbufs = rest[3:6]
    lab_refs = rest[6:6 + nsub]
    sems = rest[6 + nsub:6 + nsub + 9]
    lsems, fsems, qsems = sems[0:3], sems[3:6], sems[6:9]
    shared_s, shared_s2 = rest[6 + nsub + 9:]
    cid = lax.axis_index("c")
    sid = lax.axis_index("s")

    base = (cid * NS + sid) * b_per_w
    for j in range(nsub):
      pltpu.sync_copy(labels_hbm.at[pl.ds(base + j * sub, sub)], lab_refs[j])

    loads = [None] * nsub
    for j in range(min(3, nsub)):
      loads[j] = pltpu.async_copy(
          feats_hbm.at[pl.ds(base + j * sub, sub)], fbufs[j], lsems[j])

    _fill2d(zbuf, 0.0)

    row0 = sid * stripe
    pltpu.sync_copy(zbuf, shared_s.at[pl.ds(row0, stripe)])
    pltpu.sync_copy(zbuf, shared_s2.at[pl.ds(row0, stripe)])
    plsc.subcore_barrier()

    lane = lax.iota(jnp.int32, L)
    shift_idx = (lane + (L - 1)) & (L - 1)

    scats = [None] * nsub
    for j in range(nsub):
      b = j % 3
      loads[j].wait()

      fscat = pltpu.async_copy(fbufs[b], shared_s.at[lab_refs[j]], fsems[b],
                               add=True)

      def sq_body(r, carry, fb=fbufs[b], qb=sqbufs[b]):
        v0 = fb[r, pl.ds(0, L)]
        sq0 = v0 * v0
        sh = sq0.at[shift_idx].get(mode="promise_in_bounds")
        merged = jnp.where(lane == 1, sq0 + sh, sq0)
        qb[r, pl.ds(0, L)] = jnp.where(lane == 0, 1.0, merged)
        for c in range(1, dim // L):
          v = fb[r, pl.ds(c * L, L)]
          qb[r, pl.ds(c * L, L)] = v * v
        return carry

      lax.fori_loop(0, sub, sq_body, 0)

      scats[j] = (
          fscat,
          pltpu.async_copy(sqbufs[b], shared_s2.at[lab_refs[j]], qsems[b],
                           add=True),
      )

      if j + 3 < nsub:
        for d in scats[j]:
          d.wait()
        loads[j + 3] = pltpu.async_copy(
            feats_hbm.at[pl.ds(base + (j + 3) * sub, sub)], fbufs[b], lsems[b])

    for j in range(max(nsub - 3, 0), nsub):
      for d in scats[j]:
        d.wait()

    plsc.subcore_barrier()
    pltpu.sync_copy(shared_s.at[pl.ds(row0, stripe)],
                    s_out.at[cid, pl.ds(row0, stripe)])
    pltpu.sync_copy(shared_s2.at[pl.ds(row0, stripe)],
                    s2_out.at[cid, pl.ds(row0, stripe)])

  return sc(features, labels)


def _tc_pairwise(centers):
  k, dim = centers.shape
  f32 = jnp.float32

  def body(c_ref, m_ref):
    cmat = c_ref[:]
    sq = jnp.sum(cmat * cmat, axis=1, keepdims=True)
    gram = lax.dot_general(
        cmat, cmat, (((1,), (1,)), ((), ())),
        preferred_element_type=f32, precision=lax.Precision.HIGHEST)
    rows = lax.broadcasted_iota(jnp.int32, (k, k), 0)
    cols = lax.broadcasted_iota(jnp.int32, (k, k), 1)
    eye = rows == cols
    sq_col = jnp.sum(jnp.where(eye, sq + jnp.zeros((k, k), f32), 0.0),
                     axis=0, keepdims=True)
    d2 = jnp.maximum(sq + sq_col - 2.0 * gram, 0.0)
    pos = d2 > 0.0
    dist = jnp.where(pos, jnp.sqrt(jnp.where(pos, d2, 1.0)), 0.0)
    contrib = jnp.maximum(jnp.exp(MARGIN - dist) - 1.0, 0.0)
    m_ref[...] = jnp.where(eye, 0.0, contrib).astype(jnp.bfloat16)

  return pl.pallas_call(
      body,
      out_shape=jax.ShapeDtypeStruct((k, k), jnp.bfloat16),
      in_specs=[pl.BlockSpec((k, dim), lambda: (0, 0))],
      out_specs=pl.BlockSpec((k, k), lambda: (0, 0)),
  )(centers)


def _tc_combine(centers, pair_m, s, s2, k_pad):
  k, dim = centers.shape
  f32 = jnp.float32

  def body(c_ref, m_ref, s_ref, s2_ref, out_ref):
    cmat = c_ref[:]
    smat = (s_ref[0] + s_ref[1])[:k]
    s2mat = (s2_ref[0] + s2_ref[1])[:k]
    nvec = s2mat[:, 0:1]

    q = jnp.sum(s2mat, axis=1, keepdims=True) - nvec
    cdots = jnp.sum(cmat * smat, axis=1, keepdims=True)
    sq = jnp.sum(cmat * cmat, axis=1, keepdims=True)

    present = nvec > 0.0
    pcount = jnp.sum(present.astype(f32))
    safe_n = jnp.where(present, nvec, 1.0)
    lc_terms = jnp.where(present, (q - 2.0 * cdots + nvec * sq) / safe_n, 0.0)
    l_center = jnp.sum(lc_terms) / pcount

    presentf = present.astype(f32)
    p_bf = presentf.astype(jnp.bfloat16)
    mp = lax.dot_general(
        m_ref[...], p_bf, (((1,), (0,)), ((), ())),
        preferred_element_type=f32)
    l_spread = jnp.sum(presentf * mp)

    out_ref[0, 0] = l_center + LAMBDA_SPREAD * l_spread

  out = pl.pallas_call(
      body,
      out_shape=jax.ShapeDtypeStruct((1, 1), f32),
      in_specs=[
          pl.BlockSpec((k, dim), lambda: (0, 0)),
          pl.BlockSpec((k, k), lambda: (0, 0)),
          pl.BlockSpec((NC, k_pad, dim), lambda: (0, 0, 0)),
          pl.BlockSpec((NC, k_pad, dim), lambda: (0, 0, 0)),
      ],
      out_specs=pl.BlockSpec(memory_space=pltpu.SMEM),
  )(centers, pair_m, s, s2)
  return out[0, 0]


def kernel(features, labels, centers):
  k = centers.shape[0]
  k_pad = ((k + 8 * NS - 1) // (8 * NS)) * (8 * NS)
  labels = labels.astype(jnp.int32)
  s, s2 = _sc_segment_sums(features, labels, k_pad)
  pair_m = _tc_pairwise(centers)
  return _tc_combine(centers, pair_m, s, s2, k_pad)
